# baseline (speedup 1.0000x reference)
import functools

import jax
import jax.numpy as jnp
from jax import lax
from jax.experimental import pallas as pl
from jax.experimental.pallas import tpu as pltpu
from jax.experimental.pallas import tpu_sc as plsc

AVG = 15
MAXW = 3
GRP = AVG * MAXW
LANES = 16
CHUNK = 120


def _encoder_body(nq, n_chunks, out_per_w, subs_per_w, scale_w, scale_p,
                  src_ref, w_ref, p_ref, out_ref,
                  idx_v, pbuf, rows, pp, out_v, sem):
    nc = lax.axis_index("c")
    ns = lax.axis_index("s")
    wid = ns * 2 + nc
    tok0 = wid * (subs_per_w * AVG)
    p0 = nc * (subs_per_w * AVG)

    def p_chunk(k, _):
        pltpu.sync_copy(p_ref.at[pl.ds(p0 + k * CHUNK, CHUNK)], pbuf)

        def p_sub(u, _):
            base = u * AVG
            for q in range(nq):
                sl = pl.ds(q * LANES, LANES)
                acc = pbuf[base, sl]
                for i in range(1, AVG):
                    acc = acc + pbuf[base + i, sl]
                pp[k * (CHUNK // AVG) + u, sl] = acc * scale_p
            return 0

        lax.fori_loop(0, CHUNK // AVG, p_sub, 0)
        return 0

    lax.fori_loop(0, n_chunks, p_chunk, 0)

    neg = jnp.full((LANES,), -jnp.inf, jnp.float32)

    def o_init(j, _):
        for q in range(nq):
            out_v[j, pl.ds(q * LANES, LANES)] = neg
        return 0

    lax.fori_loop(0, out_per_w, o_init, 0)

    def w_chunk(k, _):
        pltpu.sync_copy(src_ref.at[pl.ds(tok0 + k * CHUNK, CHUNK)], idx_v)
        pltpu.async_copy(w_ref.at[idx_v], rows, sem).wait()

        def w_sub(u, _):
            sub = k * (CHUNK // AVG) + u
            j = sub // MAXW
            base = u * AVG
            for q in range(nq):
                sl = pl.ds(q * LANES, LANES)
                acc = rows[base, sl]
                for i in range(1, AVG):
                    acc = acc + rows[base + i, sl]
                e = acc * scale_w + pp[sub, sl]
                out_v[j, sl] = jnp.maximum(out_v[j, sl], e)
            return 0

        lax.fori_loop(0, CHUNK // AVG, w_sub, 0)
        return 0

    lax.fori_loop(0, n_chunks, w_chunk, 0)

    pltpu.sync_copy(out_v, out_ref.at[pl.ds(wid * out_per_w, out_per_w)])


@functools.partial(jax.jit, static_argnums=(3, 4, 5))
def _encode(src_flat, w, p, n_out, d, n_workers):
    out_per_w = n_out // n_workers
    subs_per_w = out_per_w * MAXW
    n_chunks = (subs_per_w * AVG) // CHUNK
    nq = d // LANES
    scale_w = float(d) ** 0.5 / AVG
    scale_p = 1.0 / AVG
    mesh = plsc.VectorSubcoreMesh(core_axis_name="c", subcore_axis_name="s")
    body = functools.partial(_encoder_body, nq, n_chunks, out_per_w,
                             subs_per_w, scale_w, scale_p)
    return pl.kernel(
        body,
        out_type=jax.ShapeDtypeStruct((n_out, d), jnp.float32),
        mesh=mesh,
        compiler_params=pltpu.CompilerParams(use_tc_tiling_on_sc=False),
        scratch_types=[
            pltpu.VMEM((CHUNK,), jnp.int32),
            pltpu.VMEM((CHUNK, d), jnp.float32),
            pltpu.VMEM((CHUNK, d), jnp.float32),
            pltpu.VMEM((subs_per_w, d), jnp.float32),
            pltpu.VMEM((out_per_w, d), jnp.float32),
            pltpu.SemaphoreType.DMA,
        ],
    )(src_flat, w, p)


def kernel(src, W, P):
    b, l = src.shape
    d = W.shape[1]
    n_out = b * (l // GRP)
    out = _encode(src.reshape(b * l), W, P, n_out, d, 32)
    return out.reshape(b, l // GRP, d)


# speedup vs baseline: 1.2752x; 1.2752x over previous
import functools

import jax
import jax.numpy as jnp
from jax import lax
from jax.experimental import pallas as pl
from jax.experimental.pallas import tpu as pltpu
from jax.experimental.pallas import tpu_sc as plsc

AVG = 15
MAXW = 3
GRP = AVG * MAXW
LANES = 16
CHUNK = 120
PTILES = 12


def _encoder_body(nq, n_chunks, out_per_w, subs_per_w, scale_w, scale_p,
                  src_ref, w_ref, p_ref, out_ref,
                  idx2, rows, pbuf, stage, pp, out_v, shared,
                  sem_a, sem_b):
    nc = lax.axis_index("c")
    ns = lax.axis_index("s")
    wid = ns * 2 + nc
    toks_per_w = subs_per_w * AVG
    tok0 = wid * toks_per_w
    half = n_chunks // 2 + 1

    copies = []
    for k in range(n_chunks):
        pltpu.sync_copy(src_ref.at[pl.ds(tok0 + k * CHUNK, CHUNK)],
                        idx2.at[k])
        sem = sem_a if k < half else sem_b
        copies.append(pltpu.async_copy(
            w_ref.at[idx2.at[k]], rows.at[pl.ds(k * CHUNK, CHUNK)], sem))

    g_per_t = subs_per_w // PTILES
    rows_per_t = g_per_t * AVG

    @pl.when(ns < PTILES)
    def _pool():
        p0 = nc * toks_per_w + ns * rows_per_t
        pltpu.sync_copy(p_ref.at[pl.ds(p0, rows_per_t)], pbuf)
        for t in range(g_per_t):
            base = t * AVG
            for q in range(nq):
                sl = pl.ds(q * LANES, LANES)
                acc = pbuf[base, sl]
                for i in range(1, AVG):
                    acc = acc + pbuf[base + i, sl]
                stage[t, sl] = acc * scale_p
        pltpu.sync_copy(stage, shared.at[pl.ds(ns * g_per_t, g_per_t)])

    plsc.subcore_barrier()
    pltpu.sync_copy(shared, pp)

    def compute(j, _):
        r0 = j * GRP
        res = None
        for kk in range(MAXW):
            b0 = r0 + kk * AVG
            es = []
            for q in range(nq):
                sl = pl.ds(q * LANES, LANES)
                acc = rows[b0, sl]
                for i in range(1, AVG):
                    acc = acc + rows[b0 + i, sl]
                es.append(acc * scale_w + pp[j * MAXW + kk, sl])
            if res is None:
                res = es
            else:
                res = [jnp.maximum(a, b) for a, b in zip(res, es)]
        for q in range(nq):
            out_v[j, pl.ds(q * LANES, LANES)] = res[q]
        return 0

    for k in range(half):
        copies[k].wait()
    j_mid = (half * CHUNK) // GRP
    lax.fori_loop(0, j_mid, compute, 0)

    for k in range(half, n_chunks):
        copies[k].wait()
    lax.fori_loop(j_mid, out_per_w, compute, 0)

    pltpu.sync_copy(out_v, out_ref.at[pl.ds(wid * out_per_w, out_per_w)])


@functools.partial(jax.jit, static_argnums=(3, 4, 5))
def _encode(src_flat, w, p, n_out, d, n_workers):
    out_per_w = n_out // n_workers
    subs_per_w = out_per_w * MAXW
    n_chunks = (subs_per_w * AVG) // CHUNK
    nq = d // LANES
    scale_w = float(d) ** 0.5 / AVG
    scale_p = 1.0 / AVG
    mesh = plsc.VectorSubcoreMesh(core_axis_name="c", subcore_axis_name="s")
    body = functools.partial(_encoder_body, nq, n_chunks, out_per_w,
                             subs_per_w, scale_w, scale_p)
    return pl.kernel(
        body,
        out_type=jax.ShapeDtypeStruct((n_out, d), jnp.float32),
        mesh=mesh,
        compiler_params=pltpu.CompilerParams(use_tc_tiling_on_sc=False),
        scratch_types=[
            pltpu.VMEM((n_chunks, CHUNK), jnp.int32),
            pltpu.VMEM((n_chunks * CHUNK, d), jnp.float32),
            pltpu.VMEM((subs_per_w // PTILES * AVG, d), jnp.float32),
            pltpu.VMEM((subs_per_w // PTILES, d), jnp.float32),
            pltpu.VMEM((subs_per_w, d), jnp.float32),
            pltpu.VMEM((out_per_w, d), jnp.float32),
            pltpu.VMEM_SHARED((subs_per_w, d), jnp.float32),
            pltpu.SemaphoreType.DMA,
            pltpu.SemaphoreType.DMA,
        ],
    )(src_flat, w, p)


def kernel(src, W, P):
    b, l = src.shape
    d = W.shape[1]
    n_out = b * (l // GRP)
    w_lin = lax.optimization_barrier(W.reshape(-1)).reshape(W.shape)
    out = _encode(src.reshape(b * l), w_lin, P, n_out, d, 32)
    return out.reshape(b, l // GRP, d)


# speedup vs baseline: 1.2788x; 1.0029x over previous
import functools

import jax
import jax.numpy as jnp
from jax import lax
from jax.experimental import pallas as pl
from jax.experimental.pallas import tpu as pltpu
from jax.experimental.pallas import tpu_sc as plsc

AVG = 15
MAXW = 3
GRP = AVG * MAXW
LANES = 16
CHUNK = 120
PTILES = 12


def _encoder_body(nq, n_chunks, out_per_w, subs_per_w, scale_w, scale_p,
                  src_ref, w_ref, p_ref, out_ref,
                  idx2, rows, pbuf, stage, pp, out_v, shared,
                  sem_a, sem_b):
    nc = lax.axis_index("c")
    ns = lax.axis_index("s")
    wid = ns * 2 + nc
    toks_per_w = subs_per_w * AVG
    tok0 = wid * toks_per_w
    half = n_chunks // 2 + 1

    copies = []
    for k in range(n_chunks):
        pltpu.sync_copy(src_ref.at[pl.ds(tok0 + k * CHUNK, CHUNK)],
                        idx2.at[k])
        sem = sem_a if k < half else sem_b
        copies.append(pltpu.async_copy(
            w_ref.at[idx2.at[k]], rows.at[pl.ds(k * CHUNK, CHUNK)], sem))

    g_per_t = subs_per_w // PTILES
    rows_per_t = g_per_t * AVG

    @pl.when(ns < PTILES)
    def _pool():
        p0 = nc * toks_per_w + ns * rows_per_t
        pltpu.sync_copy(p_ref.at[pl.ds(p0, rows_per_t)], pbuf)
        for t in range(g_per_t):
            base = t * AVG
            for q in range(nq):
                sl = pl.ds(q * LANES, LANES)
                acc = pbuf[base, sl]
                for i in range(1, AVG):
                    acc = acc + pbuf[base + i, sl]
                stage[t, sl] = acc * scale_p
        pltpu.sync_copy(stage, shared.at[pl.ds(ns * g_per_t, g_per_t)])

    plsc.subcore_barrier()
    pltpu.sync_copy(shared, pp)

    def compute(j, _):
        r0 = j * GRP
        res = None
        for kk in range(MAXW):
            b0 = r0 + kk * AVG
            es = []
            for q in range(nq):
                sl = pl.ds(q * LANES, LANES)
                acc = rows[b0, sl]
                for i in range(1, AVG):
                    acc = acc + rows[b0 + i, sl]
                es.append(acc * scale_w + pp[j * MAXW + kk, sl])
            if res is None:
                res = es
            else:
                res = [jnp.maximum(a, b) for a, b in zip(res, es)]
        for q in range(nq):
            out_v[j, pl.ds(q * LANES, LANES)] = res[q]
        return 0

    for k in range(half):
        copies[k].wait()
    j_mid = (half * CHUNK) // GRP
    lax.fori_loop(0, j_mid, compute, 0)

    for k in range(half, n_chunks):
        copies[k].wait()
    lax.fori_loop(j_mid, out_per_w, compute, 0)

    pltpu.sync_copy(out_v, out_ref.at[pl.ds(wid * out_per_w, out_per_w)])


@functools.partial(jax.jit, static_argnums=(3, 4, 5))
def _encode(src_flat, w, p, n_out, d, n_workers):
    out_per_w = n_out // n_workers
    subs_per_w = out_per_w * MAXW
    n_chunks = (subs_per_w * AVG) // CHUNK
    nq = d // LANES
    scale_w = float(d) ** 0.5 / AVG
    scale_p = 1.0 / AVG
    mesh = plsc.VectorSubcoreMesh(core_axis_name="c", subcore_axis_name="s")
    body = functools.partial(_encoder_body, nq, n_chunks, out_per_w,
                             subs_per_w, scale_w, scale_p)
    return pl.kernel(
        body,
        out_type=jax.ShapeDtypeStruct((n_out, d), jnp.float32),
        mesh=mesh,
        compiler_params=pltpu.CompilerParams(use_tc_tiling_on_sc=False),
        scratch_types=[
            pltpu.VMEM((n_chunks, CHUNK), jnp.int32),
            pltpu.VMEM((n_chunks * CHUNK, d), jnp.float32),
            pltpu.VMEM((subs_per_w // PTILES * AVG, d), jnp.float32),
            pltpu.VMEM((subs_per_w // PTILES, d), jnp.float32),
            pltpu.VMEM((subs_per_w, d), jnp.float32),
            pltpu.VMEM((out_per_w, d), jnp.float32),
            pltpu.VMEM_SHARED((subs_per_w, d), jnp.float32),
            pltpu.SemaphoreType.DMA,
            pltpu.SemaphoreType.DMA,
        ],
    )(src_flat, w, p)


def kernel(src, W, P):
    b, l = src.shape
    d = W.shape[1]
    n_out = b * (l // GRP)
    out = _encode(src.reshape(b * l), W, P, n_out, d, 32)
    return out.reshape(b, l // GRP, d)
